# baseline (speedup 1.0000x reference)
import functools

import jax
import jax.numpy as jnp
from jax import lax
from jax.experimental import pallas as pl
from jax.experimental.pallas import tpu as pltpu
from jax.experimental.pallas import tpu_sc as plsc

_B, _L = 4096, 50
_N = _B * _L
_D = 32
_NT = 4
_NC, _NS = 2, 16
_NW = _NC * _NS
_RPW = _N // _NW

_V = 1000000
_TAIL = _V % 128
_NSLAB = (_V - _TAIL) // 128
_SV = _V // 4

_C = 256
_NCHUNK = _RPW // _C


def _depad_body(tT0, tT1, tl0, tl1, s0, s1,
                sa, sb, ta, tb, tlv, isems, osems):
    wid = lax.axis_index("s") * _NC + lax.axis_index("c")
    slabs = (sa, sb)
    trs = (ta, tb)
    iota = lax.iota(jnp.int32, 16)
    iota4 = lax.shift_right_logical(iota, 2)
    lmod = (iota & 3) * _D

    def one_table(tT, s, tl, tsel):
        lo = wid * _NSLAB // _NW
        hi = (wid + 1) * _NSLAB // _NW
        n = hi - lo

        def fire_in(li, b):
            pltpu.async_copy(
                tT.at[:, pl.ds(128 * (lo + li), 128)], slabs[b], isems[b]
            )

        def drain_in(b):
            pltpu.make_async_copy(
                tT.at[:, pl.ds(0, 128)], slabs[b], isems[b]
            ).wait()

        def fire_out(li, b):
            pltpu.async_copy(
                trs[b], s.at[pl.ds(32 * (lo + li), 32)], osems[b]
            )

        def drain_out(b):
            pltpu.make_async_copy(
                trs[b], s.at[pl.ds(0, 32)], osems[b]
            ).wait()

        def compute(b):
            def col_grp(ic, carry):
                for cc in range(4):
                    c = ic * 4 + cc
                    lanes = lmod + c
                    for k in range(8):
                        v = slabs[b][c, pl.ds(16 * k, 16)]
                        plsc.store_scatter(
                            trs[b], [4 * k + iota4, lanes], v
                        )
                return carry

            lax.fori_loop(0, 8, col_grp, 0)

        fire_in(0, 0)
        fire_in(1, 1)

        def pair(i, carry):
            for b in range(2):
                li = 2 * i + b
                drain_in(b)

                @pl.when(i > 0)
                def _():
                    drain_out(b)

                compute(b)
                fire_out(li, b)

                @pl.when(li + 2 < n)
                def _():
                    fire_in(li + 2, b)

            return carry

        npair = n // 2
        lax.fori_loop(0, npair, pair, 0)
        for b in range(2):
            drain_out(b)

        @pl.when(n % 2 == 1)
        def _():
            drain_in(0)
            compute(0)
            fire_out(n - 1, 0)
            drain_out(0)

        @pl.when(wid == tsel)
        def _():
            pltpu.sync_copy(tl, tlv)
            for rr in range(_TAIL):
                for h in range(2):
                    trs[0][rr // 4, pl.ds(_D * (rr % 4) + 16 * h, 16)] = (
                        tlv[rr, pl.ds(16 * h, 16)]
                    )
            pltpu.sync_copy(
                trs[0].at[pl.ds(0, _TAIL // 4)],
                s.at[pl.ds((_V - _TAIL) // 4, _TAIL // 4)],
            )

    one_table(tT0, s0, tl0, 0)
    one_table(tT1, s1, tl1, 1)


def _gather_body(i0, i1, i2, i3, s0, s1, t2, t3, out,
                 x0, x1, x2, x3, q0, q1, g0, g1, sel0, sel1, r2, r3,
                 gsems, wsems):
    wid = lax.axis_index("s") * _NC + lax.axis_index("c")
    base = wid * _RPW
    ins = (i0, i1, i2, i3)
    idxs = (x0, x1, x2, x3)
    qbufs = (q0, q1)
    gbufs = (g0, g1)
    sels = (sel0, sel1)
    rows = (sel0, sel1, r2, r3)
    bigs = (s0, s1)
    smalls = (t2, t3)
    iota = lax.iota(jnp.int32, 16)

    def fire_write(ci, t):
        pltpu.async_copy(
            rows[t],
            out.at[pl.ds(base + ci * _C, _C), pl.ds(t * _D, _D)],
            wsems[t],
        )

    def drain_write(t):
        pltpu.make_async_copy(
            rows[t],
            out.at[pl.ds(base, _C), pl.ds(t * _D, _D)],
            wsems[t],
        ).wait()

    def chunk(ci, carry):
        roff = base + ci * _C
        for t in range(_NT):
            pltpu.sync_copy(ins[t].at[pl.ds(roff, _C)], idxs[t])

        for t in range(2):
            def qgrp(k, carry2):
                qbufs[t][pl.ds(16 * k, 16)] = lax.shift_right_logical(
                    idxs[t][pl.ds(16 * k, 16)], 2
                )
                return carry2

            lax.fori_loop(0, _C // 16, qgrp, 0, unroll=4)

        @pl.when(ci > 0)
        def _():
            for t in range(_NT):
                drain_write(t)

        gcopies = [
            pltpu.async_copy(bigs[t].at[qbufs[t]], gbufs[t], gsems[t])
            for t in range(2)
        ] + [
            pltpu.async_copy(smalls[t].at[idxs[2 + t]], rows[2 + t],
                             gsems[2 + t])
            for t in range(2)
        ]
        gcopies[2].wait()
        fire_write(ci, 2)
        gcopies[3].wait()
        fire_write(ci, 3)

        for t in range(2):
            gcopies[t].wait()

            def grp(g, carry2):
                rr = 16 * g + iota
                svec = (idxs[t][pl.ds(16 * g, 16)] & 3) * _D
                for j in range(_D):
                    v = plsc.load_gather(gbufs[t], [rr, svec + j])
                    plsc.store_scatter(
                        sels[t], [rr, jnp.full((16,), j, jnp.int32)], v
                    )
                return carry2

            lax.fori_loop(0, _C // 16, grp, 0)
            fire_write(ci, t)

        return carry

    lax.fori_loop(0, _NCHUNK, chunk, 0)
    for t in range(_NT):
        drain_write(t)


@jax.jit
def _run(i0, i1, i2, i3, t0, t1, t2, t3):
    mesh = plsc.VectorSubcoreMesh(core_axis_name="c", subcore_axis_name="s")

    depad = pl.kernel(
        _depad_body,
        out_type=(
            jax.ShapeDtypeStruct((_SV, 128), jnp.float32),
            jax.ShapeDtypeStruct((_SV, 128), jnp.float32),
        ),
        mesh=mesh,
        scratch_types=(
            [pltpu.VMEM((32, 128), jnp.float32) for _ in range(4)]
            + [pltpu.VMEM((_TAIL, _D), jnp.float32)]
            + [[pltpu.SemaphoreType.DMA] * 2, [pltpu.SemaphoreType.DMA] * 2]
        ),
        compiler_params=pltpu.CompilerParams(use_tc_tiling_on_sc=True, needs_layout_passes=False),
    )
    s0, s1 = depad(
        jnp.transpose(t0), jnp.transpose(t1),
        t0[_V - _TAIL:], t1[_V - _TAIL:],
    )

    gather = pl.kernel(
        _gather_body,
        out_type=jax.ShapeDtypeStruct((_N, _NT * _D), jnp.float32),
        mesh=mesh,
        scratch_types=(
            [pltpu.VMEM((_C,), jnp.int32) for _ in range(4)]
            + [pltpu.VMEM((_C,), jnp.int32) for _ in range(2)]
            + [pltpu.VMEM((_C, 128), jnp.float32) for _ in range(2)]
            + [pltpu.VMEM((_C, _D), jnp.float32) for _ in range(4)]
            + [[pltpu.SemaphoreType.DMA] * 4, [pltpu.SemaphoreType.DMA] * 4]
        ),
        compiler_params=pltpu.CompilerParams(use_tc_tiling_on_sc=False, needs_layout_passes=False),
    )
    return gather(i0, i1, i2, i3, s0, s1, t2, t3)


def kernel(input0, input1, input2, input3, table0, table1, table2, table3):
    idx = [
        jnp.transpose(x).reshape(_N)
        for x in (input0, input1, input2, input3)
    ]
    out = _run(idx[0], idx[1], idx[2], idx[3], table0, table1, table2, table3)
    return out.reshape(_L, _B, _NT * _D).transpose(1, 0, 2)


# speedup vs baseline: 1.8826x; 1.8826x over previous
import functools

import jax
import jax.numpy as jnp
from jax import lax
from jax.experimental import pallas as pl
from jax.experimental.pallas import tpu as pltpu
from jax.experimental.pallas import tpu_sc as plsc

_B, _L = 4096, 50
_N = _B * _L
_D = 32
_NT = 4
_NC, _NS = 2, 16
_NW = _NC * _NS
_RPW = _N // _NW
_G = 640
_C = 640
_NCHUNK = _RPW // _C


def _sc_body(i0, i1, i2, i3, t0, t1, t2, t3, out,
             x0, x1, x2, x3, r0, r1, r2, r3,
             isem, gs0, gs1, gs2, gs3, ws0, ws1, ws2, ws3):
    wid = lax.axis_index("s") * _NC + lax.axis_index("c")
    base = wid * _RPW
    ins = (i0, i1, i2, i3)
    tabs = (t0, t1, t2, t3)
    idxs = (x0, x1, x2, x3)
    rows = (r0, r1, r2, r3)
    gsems = (gs0, gs1, gs2, gs3)
    wsems = (ws0, ws1, ws2, ws3)

    icopies = [
        pltpu.async_copy(ins[t].at[pl.ds(base, _RPW)], idxs[t], isem)
        for t in range(_NT)
    ]
    for c in icopies:
        c.wait()

    def fire_gather(ci, t):
        pltpu.async_copy(
            tabs[t].at[idxs[t].at[pl.ds(ci * _C, _C)]],
            rows[t],
            gsems[t],
        )

    def drain_gather(t):
        pltpu.make_async_copy(
            tabs[t].at[idxs[t].at[pl.ds(0, _C)]],
            rows[t],
            gsems[t],
        ).wait()

    def fire_write(ci, t):
        pltpu.async_copy(
            rows[t],
            out.at[pl.ds(base + ci * _C, _C), pl.ds(t * _D, _D)],
            wsems[t],
        )

    def drain_write(t):
        pltpu.make_async_copy(
            rows[t],
            out.at[pl.ds(base, _C), pl.ds(t * _D, _D)],
            wsems[t],
        ).wait()

    for t in range(_NT):
        fire_gather(0, t)

    def chunk(ci, carry):
        for t in range(_NT):
            drain_gather(t)
            fire_write(ci, t)

        @pl.when(ci < _NCHUNK - 1)
        def _():
            for t in range(_NT):
                drain_write(t)
                fire_gather(ci + 1, t)

        return carry

    lax.fori_loop(0, _NCHUNK, chunk, 0)
    for t in range(_NT):
        drain_write(t)


@jax.jit
def _run(i0, i1, i2, i3, t0, t1, t2, t3):
    mesh = plsc.VectorSubcoreMesh(core_axis_name="c", subcore_axis_name="s")
    f = pl.kernel(
        _sc_body,
        out_type=jax.ShapeDtypeStruct((_N, _NT * _D), jnp.float32),
        mesh=mesh,
        scratch_types=(
            [pltpu.VMEM((_RPW,), jnp.int32) for _ in range(_NT)]
            + [pltpu.VMEM((_C, _D), jnp.float32) for _ in range(_NT)]
            + [pltpu.SemaphoreType.DMA] * 9
        ),
        compiler_params=pltpu.CompilerParams(use_tc_tiling_on_sc=False),
    )
    return f(i0, i1, i2, i3, t0, t1, t2, t3)


def kernel(input0, input1, input2, input3, table0, table1, table2, table3):
    idx = [
        jnp.transpose(x).reshape(_N)
        for x in (input0, input1, input2, input3)
    ]
    out = _run(idx[0], idx[1], idx[2], idx[3], table0, table1, table2, table3)
    return out.reshape(_L, _B, _NT * _D).transpose(1, 0, 2)
